# Initial kernel scaffold; baseline (speedup 1.0000x reference)
import jax
import jax.numpy as jnp
from jax.experimental import pallas as pl


def kernel(x, in_degree, out_degree, atom_emb, in_deg_emb, out_deg_emb, graph_token):
    raise NotImplementedError("write your pallas kernel here")


# speedup vs baseline: 1.1038x; 1.1038x over previous
import functools

import jax
import jax.numpy as jnp
from jax import lax
from jax.experimental import pallas as pl
from jax.experimental.pallas import tpu as pltpu
from jax.experimental.pallas import tpu_sc as plsc

B, N, NFEAT, H = 256, 256, 9, 768
NSRC = NFEAT + 2
LANES = 16
CHUNK = 32
HCH = H // LANES
IDXB = NSRC * CHUNK

_info = plsc.get_sparse_core_info()
NC, NS = _info.num_cores, _info.num_subcores
NW = NC * NS
GPW = B // NW
CPG = N // CHUNK


def _accum(gbuf, acc, first):
    def row(r, c):
        for k in range(HCH):
            v = gbuf[r, pl.ds(k * LANES, LANES)]
            if first:
                acc[pl.ds(r * H + k * LANES, LANES)] = v
            else:
                plsc.addupdate(acc.at[pl.ds(r * H + k * LANES, LANES)], v)
        return c
    lax.fori_loop(0, CHUNK, row, 0)


def _sc_kernel_body(idx_hbm, atom_hbm, indeg_hbm, outdeg_hbm, gtok_hbm,
                    out_hbm,
                    idx_buf, gbuf0, gbuf1, acc, gtok_v, sem0, sem1):
    wid = lax.axis_index("s") * NC + lax.axis_index("c")

    pltpu.sync_copy(gtok_hbm, gtok_v)

    def graph_body(gl, c0):
        g = wid * GPW + gl
        out_base = g * (N + 1) * H
        pltpu.sync_copy(gtok_v, out_hbm.at[pl.ds(out_base, H)])

        def chunk_body(c, cc):
            ci = g * CPG + c
            pltpu.sync_copy(idx_hbm.at[pl.ds(ci * IDXB, IDXB)], idx_buf)

            def gsrc(j):
                idx = idx_buf.at[pl.ds(j * CHUNK, CHUNK)]
                if j < NFEAT:
                    return atom_hbm.at[j].at[idx]
                if j == NFEAT:
                    return indeg_hbm.at[idx]
                return outdeg_hbm.at[idx]

            bufs = (gbuf0, gbuf1)
            sems = (sem0, sem1)
            handles = [pltpu.async_copy(gsrc(0), gbuf0, sem0), None]
            for j in range(NSRC):
                if j + 1 < NSRC:
                    handles[(j + 1) % 2] = pltpu.async_copy(
                        gsrc(j + 1), bufs[(j + 1) % 2], sems[(j + 1) % 2])
                handles[j % 2].wait()
                _accum(bufs[j % 2], acc, first=(j == 0))

            pltpu.sync_copy(
                acc, out_hbm.at[pl.ds(out_base + (1 + c * CHUNK) * H, CHUNK * H)])
            return cc
        lax.fori_loop(0, CPG, chunk_body, 0)
        return c0
    lax.fori_loop(0, GPW, graph_body, 0)


@jax.jit
def _atom_feature_sc(idx_packed, atom_emb, in_tbl, out_tbl, gtok):
    mesh = plsc.VectorSubcoreMesh(core_axis_name="c", subcore_axis_name="s")
    k = functools.partial(
        pl.kernel,
        mesh=mesh,
        out_type=jax.ShapeDtypeStruct((B * (N + 1) * H,), jnp.float32),
        scratch_types=[
            pltpu.VMEM((IDXB,), jnp.int32),
            pltpu.VMEM((CHUNK, H), jnp.float32),
            pltpu.VMEM((CHUNK, H), jnp.float32),
            pltpu.VMEM((CHUNK * H,), jnp.float32),
            pltpu.VMEM((H,), jnp.float32),
            pltpu.SemaphoreType.DMA,
            pltpu.SemaphoreType.DMA,
        ],
    )(_sc_kernel_body)
    return k(idx_packed, atom_emb, in_tbl, out_tbl, gtok)


def kernel(x, in_degree, out_degree, atom_emb, in_deg_emb, out_deg_emb, graph_token):
    T = B * N
    xt = x.reshape(T, NFEAT).T
    allidx = jnp.concatenate(
        [xt, in_degree.reshape(1, T), out_degree.reshape(1, T)], axis=0)
    idx_packed = (allidx.reshape(NSRC, T // CHUNK, CHUNK)
                  .transpose(1, 0, 2).reshape(-1))
    gtok = graph_token.reshape(H)
    out = _atom_feature_sc(idx_packed, atom_emb, in_deg_emb, out_deg_emb, gtok)
    return out.reshape(B, N + 1, H)


# speedup vs baseline: 2.6341x; 2.3864x over previous
import functools

import jax
import jax.numpy as jnp
from jax import lax
from jax.experimental import pallas as pl
from jax.experimental.pallas import tpu as pltpu
from jax.experimental.pallas import tpu_sc as plsc

B, N, NFEAT, H = 256, 256, 9, 768
NSRC = NFEAT + 2
LANES = 16
CHUNK = 32
HCH = H // LANES
IDXB = NSRC * CHUNK
GRP = 8

_info = plsc.get_sparse_core_info()
NC, NS = _info.num_cores, _info.num_subcores
NW = NC * NS
GPW = B // NW
CPG = N // CHUNK
CPW = GPW * CPG


def _accum(gbuf, acc, first):
    def row(r, c):
        for k0 in range(0, HCH, GRP):
            vs = [gbuf[r, pl.ds((k0 + u) * LANES, LANES)] for u in range(GRP)]
            for u in range(GRP):
                off = r * H + (k0 + u) * LANES
                if first:
                    acc[pl.ds(off, LANES)] = vs[u]
                else:
                    plsc.addupdate(acc.at[pl.ds(off, LANES)], vs[u])
        return c
    lax.fori_loop(0, CHUNK, row, 0)


def _sc_kernel_body(idx_hbm, atom_hbm, indeg_hbm, outdeg_hbm, gtok_hbm,
                    out_hbm,
                    idxb0, idxb1, gbuf0, gbuf1, acc0, acc1, gtok_v,
                    semi0, semi1, semg0, semg1, semo0, semo1):
    wid = lax.axis_index("s") * NC + lax.axis_index("c")
    base_ci = wid * CPW

    idxbs = (idxb0, idxb1)
    semis = (semi0, semi1)
    gbufs = (gbuf0, gbuf1)
    semgs = (semg0, semg1)
    accs = (acc0, acc1)
    semos = (semo0, semo1)

    pltpu.sync_copy(gtok_hbm, gtok_v)
    for gl in range(GPW):
        g = wid * GPW + gl
        pltpu.sync_copy(gtok_v, out_hbm.at[pl.ds(g * (N + 1) * H, H)])

    pltpu.async_copy(idx_hbm.at[pl.ds((base_ci + 0) * IDXB, IDXB)], idxb0, semi0)
    pltpu.async_copy(idx_hbm.at[pl.ds((base_ci + 1) * IDXB, IDXB)], idxb1, semi1)

    def pair_body(cc, carry):
        for p in range(2):
            ci = cc * 2 + p
            gci = base_ci + ci
            idxb = idxbs[p]
            acc = accs[p]
            out_off = ((wid * GPW + ci // CPG) * (N + 1)
                       + 1 + (ci % CPG) * CHUNK) * H

            pltpu.make_async_copy(
                idx_hbm.at[pl.ds(gci * IDXB, IDXB)], idxb, semis[p]).wait()

            @pl.when(cc > 0)
            def _():
                pltpu.make_async_copy(
                    acc, out_hbm.at[pl.ds(out_off, CHUNK * H)], semos[p]).wait()

            def gsrc(j):
                idx = idxb.at[pl.ds(j * CHUNK, CHUNK)]
                if j < NFEAT:
                    return atom_hbm.at[j].at[idx]
                if j == NFEAT:
                    return indeg_hbm.at[idx]
                return outdeg_hbm.at[idx]

            handles = [pltpu.async_copy(gsrc(0), gbuf0, semg0), None]
            for j in range(NSRC):
                if j + 1 < NSRC:
                    handles[(j + 1) % 2] = pltpu.async_copy(
                        gsrc(j + 1), gbufs[(j + 1) % 2], semgs[(j + 1) % 2])
                handles[j % 2].wait()
                if j == NSRC - 1:
                    @pl.when(ci + 2 < CPW)
                    def _():
                        pltpu.async_copy(
                            idx_hbm.at[pl.ds((gci + 2) * IDXB, IDXB)],
                            idxb, semis[p])
                _accum(gbufs[j % 2], acc, first=(j == 0))

            pltpu.async_copy(acc, out_hbm.at[pl.ds(out_off, CHUNK * H)], semos[p])
        return carry

    lax.fori_loop(0, CPW // 2, pair_body, 0)

    for p in range(2):
        pltpu.make_async_copy(
            accs[p], out_hbm.at[pl.ds(p * CHUNK * H, CHUNK * H)],
            semos[p]).wait()


@jax.jit
def _atom_feature_sc(idx_packed, atom_emb, in_tbl, out_tbl, gtok):
    mesh = plsc.VectorSubcoreMesh(core_axis_name="c", subcore_axis_name="s")
    k = functools.partial(
        pl.kernel,
        mesh=mesh,
        out_type=jax.ShapeDtypeStruct((B * (N + 1) * H,), jnp.float32),
        scratch_types=[
            pltpu.VMEM((IDXB,), jnp.int32),
            pltpu.VMEM((IDXB,), jnp.int32),
            pltpu.VMEM((CHUNK, H), jnp.float32),
            pltpu.VMEM((CHUNK, H), jnp.float32),
            pltpu.VMEM((CHUNK * H,), jnp.float32),
            pltpu.VMEM((CHUNK * H,), jnp.float32),
            pltpu.VMEM((H,), jnp.float32),
            pltpu.SemaphoreType.DMA,
            pltpu.SemaphoreType.DMA,
            pltpu.SemaphoreType.DMA,
            pltpu.SemaphoreType.DMA,
            pltpu.SemaphoreType.DMA,
            pltpu.SemaphoreType.DMA,
        ],
    )(_sc_kernel_body)
    return k(idx_packed, atom_emb, in_tbl, out_tbl, gtok)


def kernel(x, in_degree, out_degree, atom_emb, in_deg_emb, out_deg_emb, graph_token):
    T = B * N
    xt = x.reshape(T, NFEAT).T
    allidx = jnp.concatenate(
        [xt, in_degree.reshape(1, T), out_degree.reshape(1, T)], axis=0)
    idx_packed = (allidx.reshape(NSRC, T // CHUNK, CHUNK)
                  .transpose(1, 0, 2).reshape(-1))
    gtok = graph_token.reshape(H)
    out = _atom_feature_sc(idx_packed, atom_emb, in_deg_emb, out_deg_emb, gtok)
    return out.reshape(B, N + 1, H)


# speedup vs baseline: 2.6577x; 1.0090x over previous
import functools

import jax
import jax.numpy as jnp
from jax import lax
from jax.experimental import pallas as pl
from jax.experimental.pallas import tpu as pltpu
from jax.experimental.pallas import tpu_sc as plsc

B, N, NFEAT, H = 256, 256, 9, 768
NSRC = NFEAT + 2
LANES = 16
CHUNK = 32
HCH = H // LANES
GRP = 8

_info = plsc.get_sparse_core_info()
NC, NS = _info.num_cores, _info.num_subcores
NW = NC * NS
GPW = B // NW
CPG = N // CHUNK
CPW = GPW * CPG


def _accum(gbuf, acc, first):
    def row(r, c):
        for k0 in range(0, HCH, GRP):
            vs = [gbuf[r, pl.ds((k0 + u) * LANES, LANES)] for u in range(GRP)]
            for u in range(GRP):
                off = r * H + (k0 + u) * LANES
                if first:
                    acc[pl.ds(off, LANES)] = vs[u]
                else:
                    plsc.addupdate(acc.at[pl.ds(off, LANES)], vs[u])
        return c
    lax.fori_loop(0, CHUNK, row, 0)


def _sc_kernel_body(x_hbm, ind_hbm, outd_hbm, atom_hbm, indeg_hbm, outdeg_hbm,
                    gtok_hbm, out_hbm,
                    xc0, xc1, idxb, indall, outdall, gb0, gb1, acc0, acc1,
                    gtok_v,
                    semx0, semx1, semg0, semg1, semo0, semo1):
    wid = lax.axis_index("s") * NC + lax.axis_index("c")
    b0 = wid * GPW
    tok_base = b0 * N

    xcs = (xc0, xc1)
    semxs = (semx0, semx1)
    gbufs = (gb0, gb1)
    semgs = (semg0, semg1)
    accs = (acc0, acc1)
    semos = (semo0, semo1)

    pltpu.sync_copy(gtok_hbm, gtok_v)
    for gl in range(GPW):
        g = b0 + gl
        pltpu.sync_copy(gtok_v, out_hbm.at[pl.ds(g * (N + 1) * H, H)])
    pltpu.sync_copy(ind_hbm.at[pl.ds(b0, GPW)], indall)
    pltpu.sync_copy(outd_hbm.at[pl.ds(b0, GPW)], outdall)

    lane = lax.iota(jnp.int32, LANES)

    pltpu.async_copy(x_hbm.at[pl.ds(tok_base, CHUNK)], xc0, semx0)
    pltpu.async_copy(x_hbm.at[pl.ds(tok_base + CHUNK, CHUNK)], xc1, semx1)

    def pair_body(cc, carry):
        for p in range(2):
            ci = cc * 2 + p
            tok0 = tok_base + ci * CHUNK
            gl = ci // CPG
            n0 = (ci % CPG) * CHUNK
            xc = xcs[p]
            acc = accs[p]
            out_off = ((b0 + gl) * (N + 1) + 1 + n0) * H

            pltpu.make_async_copy(
                x_hbm.at[pl.ds(0, CHUNK)], xc, semxs[p]).wait()

            for j in range(NFEAT):
                colj = jnp.full((LANES,), j, jnp.int32)
                for h in range(CHUNK // LANES):
                    v = plsc.load_gather(xc, [lane + h * LANES, colj])
                    idxb[pl.ds(j * CHUNK + h * LANES, LANES)] = v

            @pl.when(cc > 0)
            def _():
                pltpu.make_async_copy(
                    acc, out_hbm.at[pl.ds(out_off, CHUNK * H)], semos[p]).wait()

            def gsrc(j):
                if j < NFEAT:
                    return atom_hbm.at[j].at[idxb.at[pl.ds(j * CHUNK, CHUNK)]]
                if j == NFEAT:
                    return indeg_hbm.at[indall.at[gl].at[pl.ds(n0, CHUNK)]]
                return outdeg_hbm.at[outdall.at[gl].at[pl.ds(n0, CHUNK)]]

            handles = [pltpu.async_copy(gsrc(0), gb0, semg0), None]
            for j in range(NSRC):
                if j + 1 < NSRC:
                    handles[(j + 1) % 2] = pltpu.async_copy(
                        gsrc(j + 1), gbufs[(j + 1) % 2], semgs[(j + 1) % 2])
                handles[j % 2].wait()
                if j == NSRC - 1:
                    @pl.when(ci + 2 < CPW)
                    def _():
                        pltpu.async_copy(
                            x_hbm.at[pl.ds(tok0 + 2 * CHUNK, CHUNK)], xc,
                            semxs[p])
                _accum(gbufs[j % 2], acc, first=(j == 0))

            pltpu.async_copy(acc, out_hbm.at[pl.ds(out_off, CHUNK * H)], semos[p])
        return carry

    lax.fori_loop(0, CPW // 2, pair_body, 0)

    for p in range(2):
        pltpu.make_async_copy(
            accs[p], out_hbm.at[pl.ds(p * CHUNK * H, CHUNK * H)],
            semos[p]).wait()


@jax.jit
def _atom_feature_sc(x2, ind, outd, atom_emb, in_tbl, out_tbl, gtok):
    mesh = plsc.VectorSubcoreMesh(core_axis_name="c", subcore_axis_name="s")
    k = functools.partial(
        pl.kernel,
        mesh=mesh,
        out_type=jax.ShapeDtypeStruct((B * (N + 1) * H,), jnp.float32),
        compiler_params=pltpu.CompilerParams(needs_layout_passes=False),
        scratch_types=[
            pltpu.VMEM((CHUNK, NFEAT), jnp.int32),
            pltpu.VMEM((CHUNK, NFEAT), jnp.int32),
            pltpu.VMEM((NFEAT * CHUNK,), jnp.int32),
            pltpu.VMEM((GPW, N), jnp.int32),
            pltpu.VMEM((GPW, N), jnp.int32),
            pltpu.VMEM((CHUNK, H), jnp.float32),
            pltpu.VMEM((CHUNK, H), jnp.float32),
            pltpu.VMEM((CHUNK * H,), jnp.float32),
            pltpu.VMEM((CHUNK * H,), jnp.float32),
            pltpu.VMEM((H,), jnp.float32),
            pltpu.SemaphoreType.DMA,
            pltpu.SemaphoreType.DMA,
            pltpu.SemaphoreType.DMA,
            pltpu.SemaphoreType.DMA,
            pltpu.SemaphoreType.DMA,
            pltpu.SemaphoreType.DMA,
        ],
    )(_sc_kernel_body)
    return k(x2, ind, outd, atom_emb, in_tbl, out_tbl, gtok)


def kernel(x, in_degree, out_degree, atom_emb, in_deg_emb, out_deg_emb, graph_token):
    x2 = x.reshape(B * N, NFEAT)
    gtok = graph_token.reshape(H)
    out = _atom_feature_sc(x2, in_degree, out_degree, atom_emb, in_deg_emb,
                           out_deg_emb, gtok)
    return out.reshape(B, N + 1, H)


# speedup vs baseline: 2.8544x; 1.0740x over previous
import functools

import jax
import jax.numpy as jnp
from jax import lax
from jax.experimental import pallas as pl
from jax.experimental.pallas import tpu as pltpu
from jax.experimental.pallas import tpu_sc as plsc

B, N, NFEAT, H = 256, 256, 9, 768
NSRC = NFEAT + 2
LANES = 16
CHUNK = 32
HCH = H // LANES
GRP = 8

_info = plsc.get_sparse_core_info()
NC, NS = _info.num_cores, _info.num_subcores
NW = NC * NS
GPW = B // NW
CPG = N // CHUNK
CPW = GPW * CPG


def _accum(gbuf, acc, carry, first):
    def row(r, c):
        for k0 in range(0, HCH, GRP):
            vs = [gbuf[r, pl.ds((k0 + u) * LANES, LANES)] for u in range(GRP)]
            for u in range(GRP):
                sl = pl.ds((k0 + u) * LANES, LANES)
                if first:
                    acc[r + 1, sl] = vs[u]
                else:
                    plsc.addupdate(acc.at[r + 1, sl], vs[u])
        return c
    lax.fori_loop(0, CHUNK - 1, row, 0)
    for k0 in range(0, HCH, GRP):
        vs = [gbuf[CHUNK - 1, pl.ds((k0 + u) * LANES, LANES)]
              for u in range(GRP)]
        for u in range(GRP):
            sl = pl.ds((k0 + u) * LANES, LANES)
            if first:
                carry[0, sl] = vs[u]
            else:
                plsc.addupdate(carry.at[0, sl], vs[u])


def _sc_kernel_body(x_hbm, ind_hbm, outd_hbm, atom_hbm, indeg_hbm, outdeg_hbm,
                    gtok_hbm, out_hbm,
                    xc0, xc1, idxb, indall, outdall, gb0, gb1, acc0, acc1,
                    carry, gtok_v,
                    semx0, semx1, semg0, semg1, semo0, semo1):
    wid = lax.axis_index("s") * NC + lax.axis_index("c")
    b0 = wid * GPW
    tok_base = b0 * N

    xcs = (xc0, xc1)
    semxs = (semx0, semx1)
    gbufs = (gb0, gb1)
    semgs = (semg0, semg1)
    accs = (acc0, acc1)
    semos = (semo0, semo1)

    pltpu.sync_copy(gtok_hbm, gtok_v)
    pltpu.sync_copy(ind_hbm.at[pl.ds(b0, GPW)], indall)
    pltpu.sync_copy(outd_hbm.at[pl.ds(b0, GPW)], outdall)

    lane = lax.iota(jnp.int32, LANES)

    pltpu.async_copy(x_hbm.at[pl.ds(tok_base, CHUNK)], xc0, semx0)
    pltpu.async_copy(x_hbm.at[pl.ds(tok_base + CHUNK, CHUNK)], xc1, semx1)

    def pair_body(cc, carry_c):
        for p in range(2):
            ci = cc * 2 + p
            tok0 = tok_base + ci * CHUNK
            gl = ci // CPG
            c = ci % CPG
            n0 = c * CHUNK
            xc = xcs[p]
            acc = accs[p]

            pltpu.make_async_copy(
                x_hbm.at[pl.ds(0, CHUNK)], xc, semxs[p]).wait()

            for j in range(NFEAT):
                colj = jnp.full((LANES,), j, jnp.int32)
                for h in range(CHUNK // LANES):
                    v = plsc.load_gather(xc, [lane + h * LANES, colj])
                    idxb[pl.ds(j * CHUNK + h * LANES, LANES)] = v

            @pl.when(cc > 0)
            def _():
                pltpu.make_async_copy(
                    acc, out_hbm.at[0].at[pl.ds(0, CHUNK)], semos[p]).wait()

            @pl.when(c == 0)
            def _():
                for k in range(HCH):
                    sl = pl.ds(k * LANES, LANES)
                    acc[0, sl] = gtok_v[0, sl]

            @pl.when(c > 0)
            def _():
                for k in range(HCH):
                    sl = pl.ds(k * LANES, LANES)
                    acc[0, sl] = carry[0, sl]

            def gsrc(j):
                if j < NFEAT:
                    return atom_hbm.at[j].at[idxb.at[pl.ds(j * CHUNK, CHUNK)]]
                if j == NFEAT:
                    return indeg_hbm.at[indall.at[gl].at[pl.ds(n0, CHUNK)]]
                return outdeg_hbm.at[outdall.at[gl].at[pl.ds(n0, CHUNK)]]

            handles = [pltpu.async_copy(gsrc(0), gb0, semg0), None]
            for j in range(NSRC):
                if j + 1 < NSRC:
                    handles[(j + 1) % 2] = pltpu.async_copy(
                        gsrc(j + 1), gbufs[(j + 1) % 2], semgs[(j + 1) % 2])
                handles[j % 2].wait()
                if j == NSRC - 1:
                    @pl.when(ci + 2 < CPW)
                    def _():
                        pltpu.async_copy(
                            x_hbm.at[pl.ds(tok0 + 2 * CHUNK, CHUNK)], xc,
                            semxs[p])
                _accum(gbufs[j % 2], acc, carry, first=(j == 0))

            pltpu.async_copy(
                acc, out_hbm.at[b0 + gl].at[pl.ds(n0, CHUNK)], semos[p])

            @pl.when(c == CPG - 1)
            def _():
                pltpu.sync_copy(
                    carry, out_hbm.at[b0 + gl].at[pl.ds(N, 1)])
        return carry_c

    lax.fori_loop(0, CPW // 2, pair_body, 0)

    for p in range(2):
        pltpu.make_async_copy(
            accs[p], out_hbm.at[0].at[pl.ds(0, CHUNK)], semos[p]).wait()


@jax.jit
def _atom_feature_sc(x2, ind, outd, atom_emb, in_tbl, out_tbl, gtok):
    mesh = plsc.VectorSubcoreMesh(core_axis_name="c", subcore_axis_name="s")
    k = functools.partial(
        pl.kernel,
        mesh=mesh,
        out_type=jax.ShapeDtypeStruct((B, N + 1, H), jnp.float32),
        compiler_params=pltpu.CompilerParams(needs_layout_passes=False),
        scratch_types=[
            pltpu.VMEM((CHUNK, NFEAT), jnp.int32),
            pltpu.VMEM((CHUNK, NFEAT), jnp.int32),
            pltpu.VMEM((NFEAT * CHUNK,), jnp.int32),
            pltpu.VMEM((GPW, N), jnp.int32),
            pltpu.VMEM((GPW, N), jnp.int32),
            pltpu.VMEM((CHUNK, H), jnp.float32),
            pltpu.VMEM((CHUNK, H), jnp.float32),
            pltpu.VMEM((CHUNK, H), jnp.float32),
            pltpu.VMEM((CHUNK, H), jnp.float32),
            pltpu.VMEM((1, H), jnp.float32),
            pltpu.VMEM((1, H), jnp.float32),
            pltpu.SemaphoreType.DMA,
            pltpu.SemaphoreType.DMA,
            pltpu.SemaphoreType.DMA,
            pltpu.SemaphoreType.DMA,
            pltpu.SemaphoreType.DMA,
            pltpu.SemaphoreType.DMA,
        ],
    )(_sc_kernel_body)
    return k(x2, ind, outd, atom_emb, in_tbl, out_tbl, gtok)


def kernel(x, in_degree, out_degree, atom_emb, in_deg_emb, out_deg_emb, graph_token):
    x2 = x.reshape(B * N, NFEAT)
    return _atom_feature_sc(x2, in_degree, out_degree, atom_emb, in_deg_emb,
                            out_deg_emb, graph_token)


# speedup vs baseline: 2.9005x; 1.0162x over previous
import functools

import jax
import jax.numpy as jnp
from jax import lax
from jax.experimental import pallas as pl
from jax.experimental.pallas import tpu as pltpu
from jax.experimental.pallas import tpu_sc as plsc

B, N, NFEAT, H = 256, 256, 9, 768
NSRC = NFEAT + 2
LANES = 16
CHUNK = 32
HALF = CHUNK // 2
NU = 2 * NSRC
HCH = H // LANES
GRP = 8

_info = plsc.get_sparse_core_info()
NC, NS = _info.num_cores, _info.num_subcores
NW = NC * NS
GPW = B // NW
CPG = N // CHUNK
CPW = GPW * CPG


def _accum_half(gbuf, h, acc, carry, first):
    r_lo = h * HALF
    r_hi = r_lo + HALF - (1 if h == 1 else 0)

    def row(r, c):
        for k0 in range(0, HCH, GRP):
            vs = [gbuf[r, pl.ds((k0 + u) * LANES, LANES)] for u in range(GRP)]
            for u in range(GRP):
                sl = pl.ds((k0 + u) * LANES, LANES)
                if first:
                    acc[r + 1, sl] = vs[u]
                else:
                    plsc.addupdate(acc.at[r + 1, sl], vs[u])
        return c
    lax.fori_loop(r_lo, r_hi, row, 0)
    if h == 1:
        for k0 in range(0, HCH, GRP):
            vs = [gbuf[CHUNK - 1, pl.ds((k0 + u) * LANES, LANES)]
                  for u in range(GRP)]
            for u in range(GRP):
                sl = pl.ds((k0 + u) * LANES, LANES)
                if first:
                    carry[0, sl] = vs[u]
                else:
                    plsc.addupdate(carry.at[0, sl], vs[u])


def _sc_kernel_body(x_hbm, ind_hbm, outd_hbm, atom_hbm, indeg_hbm, outdeg_hbm,
                    gtok_hbm, out_hbm,
                    xc0, xc1, idxb, indall, outdall, gb0, gb1, acc0, acc1,
                    carry, gtok_v,
                    semx0, semx1, semg0, semg1, semg2, semg3, semo0, semo1):
    wid = lax.axis_index("s") * NC + lax.axis_index("c")
    b0 = wid * GPW
    tok_base = b0 * N

    xcs = (xc0, xc1)
    semxs = (semx0, semx1)
    gbufs = (gb0, gb1)
    semgs = (semg0, semg1, semg2, semg3)
    accs = (acc0, acc1)
    semos = (semo0, semo1)

    pltpu.sync_copy(gtok_hbm, gtok_v)
    pltpu.sync_copy(ind_hbm.at[pl.ds(b0, GPW)], indall)
    pltpu.sync_copy(outd_hbm.at[pl.ds(b0, GPW)], outdall)

    lane = lax.iota(jnp.int32, LANES)

    pltpu.async_copy(x_hbm.at[pl.ds(tok_base, CHUNK)], xc0, semx0)
    pltpu.async_copy(x_hbm.at[pl.ds(tok_base + CHUNK, CHUNK)], xc1, semx1)

    def pair_body(cc, carry_c):
        for p in range(2):
            ci = cc * 2 + p
            tok0 = tok_base + ci * CHUNK
            gl = ci // CPG
            c = ci % CPG
            n0 = c * CHUNK
            xc = xcs[p]
            acc = accs[p]

            pltpu.make_async_copy(
                x_hbm.at[pl.ds(0, CHUNK)], xc, semxs[p]).wait()

            for j in range(NFEAT):
                colj = jnp.full((LANES,), j, jnp.int32)
                for h in range(CHUNK // LANES):
                    v = plsc.load_gather(xc, [lane + h * LANES, colj])
                    idxb[pl.ds(j * CHUNK + h * LANES, LANES)] = v

            @pl.when(cc > 0)
            def _():
                pltpu.make_async_copy(
                    acc, out_hbm.at[0].at[pl.ds(0, CHUNK)], semos[p]).wait()

            @pl.when(c == 0)
            def _():
                for k in range(HCH):
                    sl = pl.ds(k * LANES, LANES)
                    acc[0, sl] = gtok_v[0, sl]

            @pl.when(c > 0)
            def _():
                for k in range(HCH):
                    sl = pl.ds(k * LANES, LANES)
                    acc[0, sl] = carry[0, sl]

            def atom_cp(j, jpar, h):
                src = atom_hbm.at[j].at[
                    idxb.at[pl.ds(j * CHUNK + h * HALF, HALF)]]
                dst = gbufs[jpar].at[pl.ds(h * HALF, HALF)]
                return pltpu.make_async_copy(src, dst, semgs[jpar * 2 + h])

            def deg_cp(tbl, degall, jpar, h):
                src = tbl.at[degall.at[gl].at[pl.ds(n0 + h * HALF, HALF)]]
                dst = gbufs[jpar].at[pl.ds(h * HALF, HALF)]
                return pltpu.make_async_copy(src, dst, semgs[jpar * 2 + h])

            atom_cp(0, 0, 0).start()
            atom_cp(0, 0, 1).start()
            atom_cp(1, 1, 0).start()
            atom_cp(1, 1, 1).start()
            atom_cp(0, 0, 0).wait()
            _accum_half(gb0, 0, acc, carry, True)
            atom_cp(0, 0, 1).wait()
            _accum_half(gb0, 1, acc, carry, True)

            def pair(m, cc2):
                jA = 2 * m + 1
                jB = jA + 1
                atom_cp(jB, 0, 0).start()
                atom_cp(jB, 0, 1).start()
                atom_cp(jA, 1, 0).wait()
                _accum_half(gb1, 0, acc, carry, False)
                atom_cp(jA, 1, 1).wait()
                _accum_half(gb1, 1, acc, carry, False)
                @pl.when(m < 3)
                def _():
                    atom_cp(jB + 1, 1, 0).start()
                    atom_cp(jB + 1, 1, 1).start()

                @pl.when(m == 3)
                def _():
                    deg_cp(indeg_hbm, indall, 1, 0).start()
                    deg_cp(indeg_hbm, indall, 1, 1).start()
                atom_cp(jB, 0, 0).wait()
                _accum_half(gb0, 0, acc, carry, False)
                atom_cp(jB, 0, 1).wait()
                _accum_half(gb0, 1, acc, carry, False)
                return cc2
            lax.fori_loop(0, (NFEAT - 1) // 2, pair, 0)

            @pl.when(ci + 2 < CPW)
            def _():
                pltpu.async_copy(
                    x_hbm.at[pl.ds(tok0 + 2 * CHUNK, CHUNK)], xc, semxs[p])

            deg_cp(outdeg_hbm, outdall, 0, 0).start()
            deg_cp(outdeg_hbm, outdall, 0, 1).start()
            deg_cp(indeg_hbm, indall, 1, 0).wait()
            _accum_half(gb1, 0, acc, carry, False)
            deg_cp(indeg_hbm, indall, 1, 1).wait()
            _accum_half(gb1, 1, acc, carry, False)
            deg_cp(outdeg_hbm, outdall, 0, 0).wait()
            _accum_half(gb0, 0, acc, carry, False)
            deg_cp(outdeg_hbm, outdall, 0, 1).wait()
            _accum_half(gb0, 1, acc, carry, False)

            pltpu.async_copy(
                acc, out_hbm.at[b0 + gl].at[pl.ds(n0, CHUNK)], semos[p])

            @pl.when(c == CPG - 1)
            def _():
                pltpu.sync_copy(
                    carry, out_hbm.at[b0 + gl].at[pl.ds(N, 1)])
        return carry_c

    lax.fori_loop(0, CPW // 2, pair_body, 0)

    for p in range(2):
        pltpu.make_async_copy(
            accs[p], out_hbm.at[0].at[pl.ds(0, CHUNK)], semos[p]).wait()


@jax.jit
def _atom_feature_sc(x2, ind, outd, atom_emb, in_tbl, out_tbl, gtok):
    mesh = plsc.VectorSubcoreMesh(core_axis_name="c", subcore_axis_name="s")
    k = functools.partial(
        pl.kernel,
        mesh=mesh,
        out_type=jax.ShapeDtypeStruct((B, N + 1, H), jnp.float32),
        compiler_params=pltpu.CompilerParams(needs_layout_passes=False),
        scratch_types=[
            pltpu.VMEM((CHUNK, NFEAT), jnp.int32),
            pltpu.VMEM((CHUNK, NFEAT), jnp.int32),
            pltpu.VMEM((NFEAT * CHUNK,), jnp.int32),
            pltpu.VMEM((GPW, N), jnp.int32),
            pltpu.VMEM((GPW, N), jnp.int32),
            pltpu.VMEM((CHUNK, H), jnp.float32),
            pltpu.VMEM((CHUNK, H), jnp.float32),
            pltpu.VMEM((CHUNK, H), jnp.float32),
            pltpu.VMEM((CHUNK, H), jnp.float32),
            pltpu.VMEM((1, H), jnp.float32),
            pltpu.VMEM((1, H), jnp.float32),
            pltpu.SemaphoreType.DMA,
            pltpu.SemaphoreType.DMA,
            pltpu.SemaphoreType.DMA,
            pltpu.SemaphoreType.DMA,
            pltpu.SemaphoreType.DMA,
            pltpu.SemaphoreType.DMA,
            pltpu.SemaphoreType.DMA,
            pltpu.SemaphoreType.DMA,
        ],
    )(_sc_kernel_body)
    return k(x2, ind, outd, atom_emb, in_tbl, out_tbl, gtok)


def kernel(x, in_degree, out_degree, atom_emb, in_deg_emb, out_deg_emb, graph_token):
    x2 = x.reshape(B * N, NFEAT)
    return _atom_feature_sc(x2, in_degree, out_degree, atom_emb, in_deg_emb,
                            out_deg_emb, graph_token)


# speedup vs baseline: 3.3261x; 1.1467x over previous
import functools

import jax
import jax.numpy as jnp
from jax import lax
from jax.experimental import pallas as pl
from jax.experimental.pallas import tpu as pltpu
from jax.experimental.pallas import tpu_sc as plsc

B, N, NFEAT, H = 256, 256, 9, 768
W = H // 2
NSRC = NFEAT + 2
LANES = 16
CHUNK = 32
HALF = CHUNK // 2
NU = 2 * NSRC
HCH = H // LANES
WCH = W // LANES
GRP = 8

_info = plsc.get_sparse_core_info()
NC, NS = _info.num_cores, _info.num_subcores
NW = NC * NS
GPW = B // NW
CPG = N // CHUNK
CPW = GPW * CPG


def _addrow(dst_ref, r, gbuf, gr, first):
    for g0 in range(0, WCH, GRP):
        ws = [gbuf[gr, pl.ds((g0 + u) * LANES, LANES)] for u in range(GRP)]
        for u in range(GRP):
            bf = plsc.bitcast(ws[u], jnp.bfloat16)
            a, b = plsc.unpack(bf, format=plsc.PackFormat.INTERLEAVED,
                               preferred_element_type=jnp.float32)
            sla = pl.ds((g0 + u) * LANES, LANES)
            slb = pl.ds(W + (g0 + u) * LANES, LANES)
            if first:
                dst_ref[r, sla] = a
                dst_ref[r, slb] = b
            else:
                plsc.addupdate(dst_ref.at[r, sla], a)
                plsc.addupdate(dst_ref.at[r, slb], b)


def _accum_half(gbuf, h, acc, carry, first):
    r_lo = h * HALF
    r_hi = r_lo + HALF - (1 if h == 1 else 0)

    def row(r, c):
        _addrow(acc, r + 1, gbuf, r, first)
        return c
    lax.fori_loop(r_lo, r_hi, row, 0)
    if h == 1:
        _addrow(carry, 0, gbuf, CHUNK - 1, first)


def _sc_kernel_body(x_hbm, ind_hbm, outd_hbm, atom_hbm, indeg_hbm, outdeg_hbm,
                    gtok_hbm, out_hbm,
                    xc0, xc1, idxb, indall, outdall, gb0, gb1, acc0, acc1,
                    carry, gtok_v,
                    semx0, semx1, semg0, semg1, semg2, semg3, semo0, semo1):
    wid = lax.axis_index("s") * NC + lax.axis_index("c")
    b0 = wid * GPW
    tok_base = b0 * N

    xcs = (xc0, xc1)
    semxs = (semx0, semx1)
    gbufs = (gb0, gb1)
    semgs = (semg0, semg1, semg2, semg3)
    accs = (acc0, acc1)
    semos = (semo0, semo1)

    pltpu.sync_copy(gtok_hbm, gtok_v)
    pltpu.sync_copy(ind_hbm.at[pl.ds(b0, GPW)], indall)
    pltpu.sync_copy(outd_hbm.at[pl.ds(b0, GPW)], outdall)

    lane = lax.iota(jnp.int32, LANES)

    pltpu.async_copy(x_hbm.at[pl.ds(tok_base, CHUNK)], xc0, semx0)
    pltpu.async_copy(x_hbm.at[pl.ds(tok_base + CHUNK, CHUNK)], xc1, semx1)

    def pair_body(cc, carry_c):
        for p in range(2):
            ci = cc * 2 + p
            tok0 = tok_base + ci * CHUNK
            gl = ci // CPG
            c = ci % CPG
            n0 = c * CHUNK
            xc = xcs[p]
            acc = accs[p]

            pltpu.make_async_copy(
                x_hbm.at[pl.ds(0, CHUNK)], xc, semxs[p]).wait()

            for j in range(NFEAT):
                colj = jnp.full((LANES,), j, jnp.int32)
                for h in range(CHUNK // LANES):
                    v = plsc.load_gather(xc, [lane + h * LANES, colj])
                    idxb[pl.ds(j * CHUNK + h * LANES, LANES)] = v

            @pl.when(cc > 0)
            def _():
                pltpu.make_async_copy(
                    acc, out_hbm.at[0].at[pl.ds(0, CHUNK)], semos[p]).wait()

            @pl.when(c == 0)
            def _():
                for k in range(HCH):
                    sl = pl.ds(k * LANES, LANES)
                    acc[0, sl] = gtok_v[0, sl]

            @pl.when(c > 0)
            def _():
                for k in range(HCH):
                    sl = pl.ds(k * LANES, LANES)
                    acc[0, sl] = carry[0, sl]

            def atom_cp(j, jpar, h):
                src = atom_hbm.at[j].at[
                    idxb.at[pl.ds(j * CHUNK + h * HALF, HALF)]]
                dst = gbufs[jpar].at[pl.ds(h * HALF, HALF)]
                return pltpu.make_async_copy(src, dst, semgs[jpar * 2 + h])

            def deg_cp(tbl, degall, jpar, h):
                src = tbl.at[degall.at[gl].at[pl.ds(n0 + h * HALF, HALF)]]
                dst = gbufs[jpar].at[pl.ds(h * HALF, HALF)]
                return pltpu.make_async_copy(src, dst, semgs[jpar * 2 + h])

            atom_cp(0, 0, 0).start()
            atom_cp(0, 0, 1).start()
            atom_cp(1, 1, 0).start()
            atom_cp(1, 1, 1).start()
            atom_cp(0, 0, 0).wait()
            _accum_half(gb0, 0, acc, carry, True)
            atom_cp(0, 0, 1).wait()
            _accum_half(gb0, 1, acc, carry, True)

            def pair(m, cc2):
                jA = 2 * m + 1
                jB = jA + 1
                atom_cp(jB, 0, 0).start()
                atom_cp(jB, 0, 1).start()
                atom_cp(jA, 1, 0).wait()
                _accum_half(gb1, 0, acc, carry, False)
                atom_cp(jA, 1, 1).wait()
                _accum_half(gb1, 1, acc, carry, False)
                @pl.when(m < 3)
                def _():
                    atom_cp(jB + 1, 1, 0).start()
                    atom_cp(jB + 1, 1, 1).start()

                @pl.when(m == 3)
                def _():
                    deg_cp(indeg_hbm, indall, 1, 0).start()
                    deg_cp(indeg_hbm, indall, 1, 1).start()
                atom_cp(jB, 0, 0).wait()
                _accum_half(gb0, 0, acc, carry, False)
                atom_cp(jB, 0, 1).wait()
                _accum_half(gb0, 1, acc, carry, False)
                return cc2
            lax.fori_loop(0, (NFEAT - 1) // 2, pair, 0)

            @pl.when(ci + 2 < CPW)
            def _():
                pltpu.async_copy(
                    x_hbm.at[pl.ds(tok0 + 2 * CHUNK, CHUNK)], xc, semxs[p])

            deg_cp(outdeg_hbm, outdall, 0, 0).start()
            deg_cp(outdeg_hbm, outdall, 0, 1).start()
            deg_cp(indeg_hbm, indall, 1, 0).wait()
            _accum_half(gb1, 0, acc, carry, False)
            deg_cp(indeg_hbm, indall, 1, 1).wait()
            _accum_half(gb1, 1, acc, carry, False)
            deg_cp(outdeg_hbm, outdall, 0, 0).wait()
            _accum_half(gb0, 0, acc, carry, False)
            deg_cp(outdeg_hbm, outdall, 0, 1).wait()
            _accum_half(gb0, 1, acc, carry, False)

            pltpu.async_copy(
                acc, out_hbm.at[b0 + gl].at[pl.ds(n0, CHUNK)], semos[p])

            @pl.when(c == CPG - 1)
            def _():
                pltpu.sync_copy(
                    carry, out_hbm.at[b0 + gl].at[pl.ds(N, 1)])
        return carry_c

    lax.fori_loop(0, CPW // 2, pair_body, 0)

    for p in range(2):
        pltpu.make_async_copy(
            accs[p], out_hbm.at[0].at[pl.ds(0, CHUNK)], semos[p]).wait()


@jax.jit
def _atom_feature_sc(x2, ind, outd, atom_emb, in_tbl, out_tbl, gtok):
    mesh = plsc.VectorSubcoreMesh(core_axis_name="c", subcore_axis_name="s")
    k = functools.partial(
        pl.kernel,
        mesh=mesh,
        out_type=jax.ShapeDtypeStruct((B, N + 1, H), jnp.float32),
        compiler_params=pltpu.CompilerParams(needs_layout_passes=False),
        scratch_types=[
            pltpu.VMEM((CHUNK, NFEAT), jnp.int32),
            pltpu.VMEM((CHUNK, NFEAT), jnp.int32),
            pltpu.VMEM((NFEAT * CHUNK,), jnp.int32),
            pltpu.VMEM((GPW, N), jnp.int32),
            pltpu.VMEM((GPW, N), jnp.int32),
            pltpu.VMEM((CHUNK, W), jnp.int32),
            pltpu.VMEM((CHUNK, W), jnp.int32),
            pltpu.VMEM((CHUNK, H), jnp.float32),
            pltpu.VMEM((CHUNK, H), jnp.float32),
            pltpu.VMEM((1, H), jnp.float32),
            pltpu.VMEM((1, H), jnp.float32),
            pltpu.SemaphoreType.DMA,
            pltpu.SemaphoreType.DMA,
            pltpu.SemaphoreType.DMA,
            pltpu.SemaphoreType.DMA,
            pltpu.SemaphoreType.DMA,
            pltpu.SemaphoreType.DMA,
            pltpu.SemaphoreType.DMA,
            pltpu.SemaphoreType.DMA,
        ],
    )(_sc_kernel_body)
    return k(x2, ind, outd, atom_emb, in_tbl, out_tbl, gtok)


def _pack_tbl(t):
    tb = t.astype(jnp.bfloat16)
    tp = jnp.stack([tb[..., :W], tb[..., W:]], axis=-1)
    return lax.bitcast_convert_type(tp, jnp.int32)


def kernel(x, in_degree, out_degree, atom_emb, in_deg_emb, out_deg_emb, graph_token):
    x2 = x.reshape(B * N, NFEAT)
    return _atom_feature_sc(x2, in_degree, out_degree, _pack_tbl(atom_emb),
                            _pack_tbl(in_deg_emb), _pack_tbl(out_deg_emb),
                            graph_token)


# speedup vs baseline: 3.4238x; 1.0294x over previous
import functools

import jax
import jax.numpy as jnp
from jax import lax
from jax.experimental import pallas as pl
from jax.experimental.pallas import tpu as pltpu
from jax.experimental.pallas import tpu_sc as plsc

B, N, NFEAT, H = 256, 256, 9, 768
W = H // 2
NSRC = NFEAT + 2
LANES = 16
CHUNK = 32
HALF = CHUNK // 2
NU = 2 * NSRC
HCH = H // LANES
WCH = W // LANES
GRP = 8

_info = plsc.get_sparse_core_info()
NC, NS = _info.num_cores, _info.num_subcores
NW = NC * NS
GPW = B // NW
CPG = N // CHUNK
CPW = GPW * CPG


def _addrow(dst_ref, r, gbuf, gr, first):
    for g0 in range(0, WCH, GRP):
        ws = [gbuf[gr, pl.ds((g0 + u) * LANES, LANES)] for u in range(GRP)]
        for u in range(GRP):
            bf = plsc.bitcast(ws[u], jnp.bfloat16)
            a, b = plsc.unpack(bf, format=plsc.PackFormat.INTERLEAVED,
                               preferred_element_type=jnp.float32)
            sla = pl.ds((g0 + u) * LANES, LANES)
            slb = pl.ds(W + (g0 + u) * LANES, LANES)
            if first:
                dst_ref[r, sla] = a
                dst_ref[r, slb] = b
            else:
                plsc.addupdate(dst_ref.at[r, sla], a)
                plsc.addupdate(dst_ref.at[r, slb], b)


def _accum_half(gbuf, h, acc, carry, first):
    r_lo = h * HALF
    r_hi = r_lo + HALF - (1 if h == 1 else 0)

    def row(r, c):
        _addrow(acc, r + 1, gbuf, r, first)
        return c
    lax.fori_loop(r_lo, r_hi, row, 0)
    if h == 1:
        _addrow(carry, 0, gbuf, CHUNK - 1, first)


def _sc_kernel_body(x_hbm, ind_hbm, outd_hbm, atom_hbm, indeg_hbm, outdeg_hbm,
                    gtok_hbm, out_hbm,
                    xc0, xc1, idxb, indall, outdall, gb0, gb1, acc0, acc1,
                    carry, gtok_v,
                    semx0, semx1, semg0, semg1, semg2, semg3, semo0, semo1):
    wid = lax.axis_index("s") * NC + lax.axis_index("c")
    b0 = wid * GPW
    tok_base = b0 * N

    xcs = (xc0, xc1)
    semxs = (semx0, semx1)
    gbufs = (gb0, gb1)
    semgs = (semg0, semg1, semg2, semg3)
    accs = (acc0, acc1)
    semos = (semo0, semo1)

    pltpu.sync_copy(gtok_hbm, gtok_v)
    pltpu.sync_copy(ind_hbm.at[pl.ds(b0, GPW)], indall)
    pltpu.sync_copy(outd_hbm.at[pl.ds(b0, GPW)], outdall)

    lane = lax.iota(jnp.int32, LANES)

    pltpu.async_copy(x_hbm.at[pl.ds(tok_base, CHUNK)], xc0, semx0)
    pltpu.async_copy(x_hbm.at[pl.ds(tok_base + CHUNK, CHUNK)], xc1, semx1)

    def pair_body(cc, carry_c):
        for p in range(2):
            ci = cc * 2 + p
            tok0 = tok_base + ci * CHUNK
            gl = ci // CPG
            c = ci % CPG
            n0 = c * CHUNK
            xc = xcs[p]
            acc = accs[p]

            pltpu.make_async_copy(
                x_hbm.at[pl.ds(0, CHUNK)], xc, semxs[p]).wait()

            for j in range(NFEAT):
                colj = jnp.full((LANES,), j, jnp.int32)
                for h in range(CHUNK // LANES):
                    v = plsc.load_gather(xc, [lane + h * LANES, colj])
                    idxb[pl.ds(j * CHUNK + h * LANES, LANES)] = v

            @pl.when(cc > 0)
            def _():
                pltpu.make_async_copy(
                    acc, out_hbm.at[0].at[pl.ds(0, CHUNK)], semos[p]).wait()

            @pl.when(c == 0)
            def _():
                for k in range(HCH):
                    sl = pl.ds(k * LANES, LANES)
                    acc[0, sl] = gtok_v[0, sl]

            @pl.when(c > 0)
            def _():
                for k in range(HCH):
                    sl = pl.ds(k * LANES, LANES)
                    acc[0, sl] = carry[0, sl]

            def atom_cp(j, jpar, h):
                src = atom_hbm.at[j].at[
                    idxb.at[pl.ds(j * CHUNK + h * HALF, HALF)]]
                dst = gbufs[jpar].at[pl.ds(h * HALF, HALF)]
                return pltpu.make_async_copy(src, dst, semgs[jpar * 2 + h])

            def deg_cp(tbl, degall, jpar, h):
                src = tbl.at[degall.at[gl].at[pl.ds(n0 + h * HALF, HALF)]]
                dst = gbufs[jpar].at[pl.ds(h * HALF, HALF)]
                return pltpu.make_async_copy(src, dst, semgs[jpar * 2 + h])

            atom_cp(0, 0, 0).start()
            atom_cp(0, 0, 1).start()
            atom_cp(1, 1, 0).start()
            atom_cp(1, 1, 1).start()
            atom_cp(0, 0, 0).wait()
            _accum_half(gb0, 0, acc, carry, True)
            atom_cp(0, 0, 1).wait()
            _accum_half(gb0, 1, acc, carry, True)

            def pair(m, cc2):
                jA = 2 * m + 1
                jB = jA + 1
                atom_cp(jB, 0, 0).start()
                atom_cp(jB, 0, 1).start()
                atom_cp(jA, 1, 0).wait()
                _accum_half(gb1, 0, acc, carry, False)
                atom_cp(jA, 1, 1).wait()
                _accum_half(gb1, 1, acc, carry, False)
                @pl.when(m < 3)
                def _():
                    atom_cp(jB + 1, 1, 0).start()
                    atom_cp(jB + 1, 1, 1).start()

                @pl.when(m == 3)
                def _():
                    deg_cp(indeg_hbm, indall, 1, 0).start()
                    deg_cp(indeg_hbm, indall, 1, 1).start()
                atom_cp(jB, 0, 0).wait()
                _accum_half(gb0, 0, acc, carry, False)
                atom_cp(jB, 0, 1).wait()
                _accum_half(gb0, 1, acc, carry, False)
                return cc2
            lax.fori_loop(0, (NFEAT - 1) // 2, pair, 0)

            @pl.when(ci + 2 < CPW)
            def _():
                pltpu.async_copy(
                    x_hbm.at[pl.ds(tok0 + 2 * CHUNK, CHUNK)], xc, semxs[p])

            deg_cp(outdeg_hbm, outdall, 0, 0).start()
            deg_cp(outdeg_hbm, outdall, 0, 1).start()
            deg_cp(indeg_hbm, indall, 1, 0).wait()
            _accum_half(gb1, 0, acc, carry, False)
            deg_cp(indeg_hbm, indall, 1, 1).wait()
            _accum_half(gb1, 1, acc, carry, False)
            deg_cp(outdeg_hbm, outdall, 0, 0).wait()
            _accum_half(gb0, 0, acc, carry, False)
            deg_cp(outdeg_hbm, outdall, 0, 1).wait()
            _accum_half(gb0, 1, acc, carry, False)

            pltpu.async_copy(
                acc, out_hbm.at[b0 + gl].at[pl.ds(n0, CHUNK)], semos[p])

            @pl.when(c == CPG - 1)
            def _():
                pltpu.sync_copy(
                    carry, out_hbm.at[b0 + gl].at[pl.ds(N, 1)])
        return carry_c

    lax.fori_loop(0, CPW // 2, pair_body, 0)

    for p in range(2):
        pltpu.make_async_copy(
            accs[p], out_hbm.at[0].at[pl.ds(0, CHUNK)], semos[p]).wait()


@jax.jit
def _atom_feature_sc(x2, ind, outd, atom_emb, in_tbl, out_tbl, gtok):
    mesh = plsc.VectorSubcoreMesh(core_axis_name="c", subcore_axis_name="s")
    k = functools.partial(
        pl.kernel,
        mesh=mesh,
        out_type=jax.ShapeDtypeStruct((B, N + 1, H), jnp.float32),
        compiler_params=pltpu.CompilerParams(needs_layout_passes=False),
        scratch_types=[
            pltpu.VMEM((CHUNK, NFEAT), jnp.int32),
            pltpu.VMEM((CHUNK, NFEAT), jnp.int32),
            pltpu.VMEM((NFEAT * CHUNK,), jnp.int32),
            pltpu.VMEM((GPW, N), jnp.int32),
            pltpu.VMEM((GPW, N), jnp.int32),
            pltpu.VMEM((CHUNK, W), jnp.int32),
            pltpu.VMEM((CHUNK, W), jnp.int32),
            pltpu.VMEM((CHUNK, H), jnp.float32),
            pltpu.VMEM((CHUNK, H), jnp.float32),
            pltpu.VMEM((1, H), jnp.float32),
            pltpu.VMEM((1, H), jnp.float32),
            pltpu.SemaphoreType.DMA,
            pltpu.SemaphoreType.DMA,
            pltpu.SemaphoreType.DMA,
            pltpu.SemaphoreType.DMA,
            pltpu.SemaphoreType.DMA,
            pltpu.SemaphoreType.DMA,
            pltpu.SemaphoreType.DMA,
            pltpu.SemaphoreType.DMA,
        ],
    )(_sc_kernel_body)
    return k(x2, ind, outd, atom_emb, in_tbl, out_tbl, gtok)


def _pack_tbl(t):
    bits = lax.bitcast_convert_type(t, jnp.uint32)
    r = (bits + 0x7FFF + ((bits >> 16) & 1)) >> 16
    packed = r[..., :W] | (r[..., W:] << 16)
    return lax.bitcast_convert_type(packed, jnp.int32)


def kernel(x, in_degree, out_degree, atom_emb, in_deg_emb, out_deg_emb, graph_token):
    x2 = x.reshape(B * N, NFEAT)
    return _atom_feature_sc(x2, in_degree, out_degree, _pack_tbl(atom_emb),
                            _pack_tbl(in_deg_emb), _pack_tbl(out_deg_emb),
                            graph_token)
